# baseline (speedup 1.0000x reference)
import functools

import jax
import jax.numpy as jnp
from jax import lax
from jax.experimental import pallas as pl
from jax.experimental.pallas import tpu as pltpu
from jax.experimental.pallas import tpu_sc as plsc

VOCAB = 100000
BATCH = 16384
DIM = 300
PDIM = 384

NC = 2
NS = 16
NW = NC * NS
CHUNK = 64
ROWS_PER_W = BATCH // NW
N_CHUNKS = ROWS_PER_W // CHUNK
SLOTS = 3

TAIL_COLS = 512

_mesh = plsc.VectorSubcoreMesh(core_axis_name="c", subcore_axis_name="s")


def _tail_copy_kernel(x_ref, o_ref):
    o_ref[...] = x_ref[...].T


def _build_tail(tabT):
    return pl.pallas_call(
        _tail_copy_kernel,
        grid=((VOCAB + TAIL_COLS - 1) // TAIL_COLS,),
        in_specs=[pl.BlockSpec((128, TAIL_COLS), lambda i: (2, i))],
        out_specs=pl.BlockSpec((TAIL_COLS, 128), lambda i: (i, 0)),
        out_shape=jax.ShapeDtypeStruct((VOCAB, 128), jnp.float32),
    )(tabT)


@functools.partial(
    pl.kernel,
    mesh=_mesh,
    out_type=jax.ShapeDtypeStruct((BATCH, PDIM), jnp.float32),
    scratch_types=[
        pltpu.VMEM((ROWS_PER_W,), jnp.int32),
        pltpu.VMEM((SLOTS, CHUNK, PDIM), jnp.float32),
        pltpu.SemaphoreType.DMA,
        pltpu.SemaphoreType.DMA,
    ],
)
def _gather_kernel(tab, tail, poses_hbm, out_hbm, idx_v, stage, gsem, wsem):
    wid = lax.axis_index("s") * NC + lax.axis_index("c")
    base = wid * ROWS_PER_W
    pltpu.sync_copy(poses_hbm.at[pl.ds(base, ROWS_PER_W)], idx_v)

    def gather(c):
        s = c % SLOTS
        ids = idx_v.at[pl.ds(c * CHUNK, CHUNK)]
        return [
            pltpu.async_copy(tab.at[ids],
                             stage.at[s, :, pl.ds(0, 256)], gsem),
            pltpu.async_copy(tail.at[ids],
                             stage.at[s, :, pl.ds(256, 128)], gsem),
        ]

    def put(c):
        rows = pl.ds(base + c * CHUNK, CHUNK)
        return pltpu.async_copy(stage.at[c % SLOTS], out_hbm.at[rows], wsem)

    pend_g = {c: gather(c) for c in range(min(SLOTS - 1, N_CHUNKS))}
    pend_w = {}
    for c in range(N_CHUNKS):
        for cp in pend_g.pop(c):
            cp.wait()
        if c >= 1:
            pend_w.pop(c - 1).wait()
        if c + SLOTS - 1 < N_CHUNKS:
            pend_g[c + SLOTS - 1] = gather(c + SLOTS - 1)
        pend_w[c] = put(c)
    for c in sorted(pend_w):
        pend_w[c].wait()


def kernel(pretrained, poses):
    tail = _build_tail(pretrained.T)
    tab_ab = lax.slice(pretrained, (0, 0), (VOCAB, 256))
    out_pad = _gather_kernel(tab_ab, tail, poses.astype(jnp.int32))
    return lax.slice(out_pad, (0, 0), (BATCH, DIM))


# speedup vs baseline: 1.9395x; 1.9395x over previous
import functools

import jax
import jax.numpy as jnp
from jax import lax
from jax.experimental import pallas as pl
from jax.experimental.pallas import tpu as pltpu
from jax.experimental.pallas import tpu_sc as plsc

VOCAB = 100000
BATCH = 16384
DIM = 300

NC = 2
NS = 16
NW = NC * NS
MAXK = (DIM + NW - 1) // NW
REM = DIM - (MAXK - 1) * NW
OUT_CHUNK = 4096
VEC = 16

_mesh = plsc.VectorSubcoreMesh(core_axis_name="c", subcore_axis_name="s")


@functools.partial(
    pl.kernel,
    mesh=_mesh,
    out_type=jax.ShapeDtypeStruct((DIM, BATCH), jnp.float32),
    scratch_types=[
        pltpu.VMEM((BATCH,), jnp.int32),
        pltpu.VMEM((VOCAB,), jnp.float32),
        pltpu.VMEM((2, OUT_CHUNK), jnp.float32),
        pltpu.SemaphoreType.DMA,
        pltpu.SemaphoreType.DMA,
    ],
    compiler_params=pltpu.CompilerParams(needs_layout_passes=False),
)
def _gather_rows_kernel(tabT, poses_hbm, out_hbm, poses_v, row_v, obuf,
                        rsem, wsem):
    wid = lax.axis_index("s") * NC + lax.axis_index("c")
    pltpu.sync_copy(poses_hbm, poses_v)

    def do_row(row):
        pltpu.sync_copy(tabT.at[row], row_v)
        pend = []
        for c in range(BATCH // OUT_CHUNK):
            if c >= 2:
                pend[c - 2].wait()

            def body(i, _, c=c):
                idx = poses_v[pl.ds(c * OUT_CHUNK + i * VEC, VEC)]
                obuf[c % 2, pl.ds(i * VEC, VEC)] = plsc.load_gather(
                    row_v, [idx])
                return _

            lax.fori_loop(0, OUT_CHUNK // VEC, body, None)
            pend.append(pltpu.async_copy(
                obuf.at[c % 2],
                out_hbm.at[row, pl.ds(c * OUT_CHUNK, OUT_CHUNK)], wsem))
        pend[-2].wait()
        pend[-1].wait()

    for k in range(MAXK - 1):
        do_row(wid + k * NW)

    @pl.when(wid < REM)
    def _():
        do_row(wid + (MAXK - 1) * NW)


def kernel(pretrained, poses):
    outT = _gather_rows_kernel(pretrained.T, poses.astype(jnp.int32))
    return outT.T
